# baseline (device time: 214427 ns/iter reference)
import jax
import jax.numpy as jnp
from jax import lax
from jax.experimental import pallas as pl
from jax.experimental.pallas import tpu as pltpu

N_DEV = 16

RING = [0, 4, 8, 12, 13, 9, 5, 1, 2, 6, 10, 14, 15, 11, 7, 3]
INV = [RING.index(i) for i in range(N_DEV)]


def _gelu(y):
    c = 0.7978845608028654
    return 0.5 * y * (1.0 + jnp.tanh(c * (y + 0.044715 * y * y * y)))


def kernel(x, w_mat):
    m_global, k_per = x.shape
    _, n = w_mat.shape
    m_per = m_global // N_DEV
    n_half = n // 2

    ring_arr = jnp.array(RING, dtype=jnp.int32)
    inv_arr = jnp.array(INV, dtype=jnp.int32)

    def body(ring_ref, inv_ref, x_ref, w_ref, out_ref,
             send_r, send_l, recv_r, recv_l,
             send_sem_r, recv_sem_r, send_sem_l, recv_sem_l):
        my = lax.axis_index("i")
        p = inv_ref[my]
        right = ring_ref[lax.rem(p + 1, N_DEV)]
        left = ring_ref[lax.rem(p + N_DEV - 1, N_DEV)]

        barrier_sem = pltpu.get_barrier_semaphore()
        for nbr in (left, right):
            pl.semaphore_signal(barrier_sem, inc=1, device_id=(nbr,),
                                device_id_type=pl.DeviceIdType.MESH)
        pl.semaphore_wait(barrier_sem, 2)

        def partial_r(c):
            xc = x_ref[pl.ds(c * m_per, m_per), :]
            return jnp.dot(xc, w_ref[:, :n_half],
                           preferred_element_type=jnp.float32)

        def partial_l(c):
            xc = x_ref[pl.ds(c * m_per, m_per), :]
            return jnp.dot(xc, w_ref[:, n_half:],
                           preferred_element_type=jnp.float32)

        send_r[:, :] = partial_r(left)
        send_l[:, :] = partial_l(right)

        for s in range(N_DEV - 1):
            rdma_r = pltpu.make_async_remote_copy(
                src_ref=send_r, dst_ref=recv_r.at[s],
                send_sem=send_sem_r.at[s], recv_sem=recv_sem_r.at[s],
                device_id=(right,), device_id_type=pl.DeviceIdType.MESH)
            rdma_l = pltpu.make_async_remote_copy(
                src_ref=send_l, dst_ref=recv_l.at[s],
                send_sem=send_sem_l.at[s], recv_sem=recv_sem_l.at[s],
                device_id=(left,), device_id_type=pl.DeviceIdType.MESH)
            rdma_r.start()
            rdma_l.start()
            rdma_r.wait()
            rdma_l.wait()

            c_r = ring_ref[lax.rem(p + 2 * N_DEV - 2 - s, N_DEV)]
            c_l = ring_ref[lax.rem(p + 2 + s, N_DEV)]
            if s < N_DEV - 2:
                send_r[:, :] = recv_r[s] + partial_r(c_r)
                send_l[:, :] = recv_l[s] + partial_l(c_l)
            else:
                out_ref[:, :n_half] = _gelu(recv_r[s] + partial_r(c_r))
                out_ref[:, n_half:] = _gelu(recv_l[s] + partial_l(c_l))

    out_shape = jax.ShapeDtypeStruct((m_per, n), jnp.float32)
    return pl.pallas_call(
        body,
        out_shape=out_shape,
        in_specs=[
            pl.BlockSpec(memory_space=pltpu.SMEM),
            pl.BlockSpec(memory_space=pltpu.SMEM),
            pl.BlockSpec(memory_space=pltpu.VMEM),
            pl.BlockSpec(memory_space=pltpu.VMEM),
        ],
        out_specs=pl.BlockSpec(memory_space=pltpu.VMEM),
        scratch_shapes=[
            pltpu.VMEM((m_per, n_half), jnp.float32),
            pltpu.VMEM((m_per, n_half), jnp.float32),
            pltpu.VMEM((N_DEV - 1, m_per, n_half), jnp.float32),
            pltpu.VMEM((N_DEV - 1, m_per, n_half), jnp.float32),
            pltpu.SemaphoreType.DMA((N_DEV - 1,)),
            pltpu.SemaphoreType.DMA((N_DEV - 1,)),
            pltpu.SemaphoreType.DMA((N_DEV - 1,)),
            pltpu.SemaphoreType.DMA((N_DEV - 1,)),
        ],
        compiler_params=pltpu.CompilerParams(
            collective_id=0,
            vmem_limit_bytes=100 * 1024 * 1024,
        ),
    )(ring_arr, inv_arr, x, w_mat)


# device time: 182398 ns/iter; 1.1756x vs baseline; 1.1756x over previous
import jax
import jax.numpy as jnp
from jax import lax
from jax.experimental import pallas as pl
from jax.experimental.pallas import tpu as pltpu

N_DEV = 16

RING = [0, 4, 8, 12, 13, 9, 5, 1, 2, 6, 10, 14, 15, 11, 7, 3]
INV = [RING.index(i) for i in range(N_DEV)]

N_HOPS = N_DEV - 1
FLOWS_PER_DIR = 2
N_FLOWS = 2 * FLOWS_PER_DIR


def _gelu(y):
    c = 0.7978845608028654
    return 0.5 * y * (1.0 + jnp.tanh(c * (y + 0.044715 * y * y * y)))


def kernel(x, w_mat):
    m_global, k_per = x.shape
    _, n = w_mat.shape
    m_per = m_global // N_DEV
    n_sub = n // N_FLOWS

    ring_arr = jnp.array(RING, dtype=jnp.int32)
    inv_arr = jnp.array(INV, dtype=jnp.int32)

    def body(ring_ref, inv_ref, x_ref, w_ref, out_ref, *scr):
        send_bufs = scr[0:N_FLOWS]
        recv_bufs = scr[N_FLOWS:2 * N_FLOWS]
        send_sems = scr[2 * N_FLOWS:3 * N_FLOWS]
        recv_sems = scr[3 * N_FLOWS:4 * N_FLOWS]

        my = lax.axis_index("i")
        p = inv_ref[my]
        right = ring_ref[lax.rem(p + 1, N_DEV)]
        left = ring_ref[lax.rem(p + N_DEV - 1, N_DEV)]

        barrier_sem = pltpu.get_barrier_semaphore()
        for nbr in (left, right):
            pl.semaphore_signal(barrier_sem, inc=1, device_id=(nbr,),
                                device_id_type=pl.DeviceIdType.MESH)
        pl.semaphore_wait(barrier_sem, 2)

        def col_lo(fi):
            return fi * n_sub

        def is_right(fi):
            return fi < FLOWS_PER_DIR

        def partial(fi, c):
            xc = x_ref[pl.ds(c * m_per, m_per), :]
            lo = col_lo(fi)
            return jnp.dot(xc, w_ref[:, lo:lo + n_sub],
                           preferred_element_type=jnp.float32)

        def chunk_sent(fi, s):
            if is_right(fi):
                return ring_ref[lax.rem(p + 2 * N_DEV - 1 - s, N_DEV)]
            return ring_ref[lax.rem(p + 1 + s, N_DEV)]

        def chunk_recvd(fi, s):
            if is_right(fi):
                return ring_ref[lax.rem(p + 2 * N_DEV - 2 - s, N_DEV)]
            return ring_ref[lax.rem(p + 2 + s, N_DEV)]

        descs = {}

        def start(fi, s):
            tgt = right if is_right(fi) else left
            rdma = pltpu.make_async_remote_copy(
                src_ref=send_bufs[fi].at[s % 2],
                dst_ref=recv_bufs[fi].at[s],
                send_sem=send_sems[fi].at[s % 2],
                recv_sem=recv_sems[fi].at[s],
                device_id=(tgt,), device_id_type=pl.DeviceIdType.MESH)
            rdma.start()
            descs[(fi, s)] = rdma

        for fi in range(N_FLOWS):
            send_bufs[fi][0] = partial(fi, chunk_sent(fi, 0))
            start(fi, 0)

        flow_order = (0, 2, 1, 3)

        for s in range(N_HOPS):
            for fi in flow_order:
                descs[(fi, s)].wait_recv()
                c = chunk_recvd(fi, s)
                if s < N_HOPS - 1:
                    if s >= 1:
                        descs[(fi, s - 1)].wait_send()
                    send_bufs[fi][(s + 1) % 2] = (
                        recv_bufs[fi][s] + partial(fi, c))
                    start(fi, s + 1)
                else:
                    lo = col_lo(fi)
                    out_ref[:, lo:lo + n_sub] = _gelu(
                        recv_bufs[fi][s] + partial(fi, c))

        for fi in range(N_FLOWS):
            for s in (N_HOPS - 2, N_HOPS - 1):
                descs[(fi, s)].wait_send()

    out_shape = jax.ShapeDtypeStruct((m_per, n), jnp.float32)
    scratch = (
        [pltpu.VMEM((2, m_per, n_sub), jnp.float32)] * N_FLOWS +
        [pltpu.VMEM((N_HOPS, m_per, n_sub), jnp.float32)] * N_FLOWS +
        [pltpu.SemaphoreType.DMA((2,))] * N_FLOWS +
        [pltpu.SemaphoreType.DMA((N_HOPS,))] * N_FLOWS
    )
    return pl.pallas_call(
        body,
        out_shape=out_shape,
        in_specs=[
            pl.BlockSpec(memory_space=pltpu.SMEM),
            pl.BlockSpec(memory_space=pltpu.SMEM),
            pl.BlockSpec(memory_space=pltpu.VMEM),
            pl.BlockSpec(memory_space=pltpu.VMEM),
        ],
        out_specs=pl.BlockSpec(memory_space=pltpu.VMEM),
        scratch_shapes=scratch,
        compiler_params=pltpu.CompilerParams(
            collective_id=0,
            vmem_limit_bytes=100 * 1024 * 1024,
        ),
    )(ring_arr, inv_arr, x, w_mat)


# device time: 182184 ns/iter; 1.1770x vs baseline; 1.0012x over previous
import jax
import jax.numpy as jnp
from jax import lax
from jax.experimental import pallas as pl
from jax.experimental.pallas import tpu as pltpu

N_DEV = 16

RING = [0, 4, 8, 12, 13, 9, 5, 1, 2, 6, 10, 14, 15, 11, 7, 3]
INV = [RING.index(i) for i in range(N_DEV)]

N_HOPS = N_DEV - 1
FLOWS_PER_DIR = 2
N_FLOWS = 2 * FLOWS_PER_DIR


def _gelu(y):
    c = 0.7978845608028654
    return 0.5 * y * (1.0 + jnp.tanh(c * (y + 0.044715 * y * y * y)))


def kernel(x, w_mat):
    m_global, k_per = x.shape
    _, n = w_mat.shape
    m_per = m_global // N_DEV
    n_sub = n // N_FLOWS

    ring_arr = jnp.array(RING, dtype=jnp.int32)
    inv_arr = jnp.array(INV, dtype=jnp.int32)

    def body(ring_ref, inv_ref, x_ref, w_ref, out_ref, *scr):
        send_bufs = scr[0:N_FLOWS]
        recv_bufs = scr[N_FLOWS:2 * N_FLOWS]
        send_sems = scr[2 * N_FLOWS:3 * N_FLOWS]
        recv_sems = scr[3 * N_FLOWS:4 * N_FLOWS]

        my = lax.axis_index("i")
        p = inv_ref[my]
        right = ring_ref[lax.rem(p + 1, N_DEV)]
        left = ring_ref[lax.rem(p + N_DEV - 1, N_DEV)]

        barrier_sem = pltpu.get_barrier_semaphore()
        for nbr in (left, right):
            pl.semaphore_signal(barrier_sem, inc=1, device_id=(nbr,),
                                device_id_type=pl.DeviceIdType.MESH)
        pl.semaphore_wait(barrier_sem, 2)

        def col_lo(fi):
            return fi * n_sub

        def is_right(fi):
            return fi < FLOWS_PER_DIR

        def partial(fi, c):
            xc = x_ref[pl.ds(c * m_per, m_per), :]
            lo = col_lo(fi)
            return jnp.dot(xc, w_ref[:, lo:lo + n_sub],
                           preferred_element_type=jnp.float32)

        def chunk_sent(fi, s):
            if is_right(fi):
                return ring_ref[lax.rem(p + 2 * N_DEV - 1 - s, N_DEV)]
            return ring_ref[lax.rem(p + 1 + s, N_DEV)]

        def chunk_recvd(fi, s):
            if is_right(fi):
                return ring_ref[lax.rem(p + 2 * N_DEV - 2 - s, N_DEV)]
            return ring_ref[lax.rem(p + 2 + s, N_DEV)]

        descs = {}

        def start(fi, s):
            tgt = right if is_right(fi) else left
            rdma = pltpu.make_async_remote_copy(
                src_ref=send_bufs[fi].at[s % 2],
                dst_ref=recv_bufs[fi].at[s],
                send_sem=send_sems[fi].at[s % 2],
                recv_sem=recv_sems[fi].at[s],
                device_id=(tgt,), device_id_type=pl.DeviceIdType.MESH)
            rdma.start()
            descs[(fi, s)] = rdma

        for fi in range(N_FLOWS):
            send_bufs[fi][0] = partial(fi, chunk_sent(fi, 0))
            start(fi, 0)

        flow_order = (0, 2, 1, 3)

        for s in range(N_HOPS):
            for fi in flow_order:
                if s >= 1:
                    descs[(fi, s - 1)].wait_send()
                c = chunk_recvd(fi, s)
                send_bufs[fi][(s + 1) % 2] = partial(fi, c)
            for fi in flow_order:
                descs[(fi, s)].wait_recv()
                if s < N_HOPS - 1:
                    slot = (s + 1) % 2
                    send_bufs[fi][slot] = (
                        send_bufs[fi][slot] + recv_bufs[fi][s])
                    start(fi, s + 1)
                else:
                    lo = col_lo(fi)
                    out_ref[:, lo:lo + n_sub] = _gelu(
                        send_bufs[fi][(s + 1) % 2] + recv_bufs[fi][s])

        for fi in range(N_FLOWS):
            descs[(fi, N_HOPS - 1)].wait_send()

    out_shape = jax.ShapeDtypeStruct((m_per, n), jnp.float32)
    scratch = (
        [pltpu.VMEM((2, m_per, n_sub), jnp.float32)] * N_FLOWS +
        [pltpu.VMEM((N_HOPS, m_per, n_sub), jnp.float32)] * N_FLOWS +
        [pltpu.SemaphoreType.DMA((2,))] * N_FLOWS +
        [pltpu.SemaphoreType.DMA((N_HOPS,))] * N_FLOWS
    )
    return pl.pallas_call(
        body,
        out_shape=out_shape,
        in_specs=[
            pl.BlockSpec(memory_space=pltpu.SMEM),
            pl.BlockSpec(memory_space=pltpu.SMEM),
            pl.BlockSpec(memory_space=pltpu.VMEM),
            pl.BlockSpec(memory_space=pltpu.VMEM),
        ],
        out_specs=pl.BlockSpec(memory_space=pltpu.VMEM),
        scratch_shapes=scratch,
        compiler_params=pltpu.CompilerParams(
            collective_id=0,
            vmem_limit_bytes=100 * 1024 * 1024,
        ),
    )(ring_arr, inv_arr, x, w_mat)
